# baseline (speedup 1.0000x reference)
import functools

import jax
import jax.numpy as jnp
from jax import lax
from jax.experimental import pallas as pl
from jax.experimental.pallas import tpu as pltpu
from jax.experimental.pallas import tpu_sc as plsc


def _sc_gather(X, idx_flat):
    R = idx_flat.shape[0]
    D = X.shape[1]
    info = plsc.get_sparse_core_info()
    NW = info.num_cores * info.num_subcores
    rows_per_w = R // NW
    CH = 128
    n_ch = rows_per_w // CH
    mesh = plsc.VectorSubcoreMesh(core_axis_name="c", subcore_axis_name="s")

    @functools.partial(
        pl.kernel,
        mesh=mesh,
        out_type=jax.ShapeDtypeStruct((R, D), jnp.float32),
        scratch_types=[
            pltpu.VMEM((CH,), jnp.int32),
            pltpu.VMEM((CH, D), jnp.float32),
            pltpu.SemaphoreType.DMA,
        ],
    )
    def gather_k(x_hbm, idx_hbm, out_hbm, idx_v, rows_v, sem):
        c = lax.axis_index("c")
        s = lax.axis_index("s")
        wid = s * info.num_cores + c
        base = wid * rows_per_w

        def body(i, carry):
            off = base + i * CH
            pltpu.sync_copy(idx_hbm.at[pl.ds(off, CH)], idx_v)
            pltpu.async_copy(x_hbm.at[idx_v], rows_v, sem).wait()
            pltpu.sync_copy(rows_v, out_hbm.at[pl.ds(off, CH)])
            return carry

        lax.fori_loop(0, n_ch, body, 0)

    return gather_k(X, idx_flat)


def _tc_attention(xg, W1, b1, W2v):
    E, S, D = xg.shape
    H = W1.shape[2]
    b1 = b1.reshape(E, 1, H)
    W2v = W2v.reshape(E, 1, H)

    def body(x_ref, w1_ref, b1_ref, w2_ref, z_ref, beta_ref):
        x = x_ref[0]
        hh = jnp.dot(x, w1_ref[0], preferred_element_type=jnp.float32)
        hh = hh + b1_ref[0]
        hh = jnp.where(hh >= 0, hh, 0.01 * hh)
        logits = jnp.sum(hh * w2_ref[0], axis=1)
        m = jnp.max(logits)
        e = jnp.exp(logits - m)
        beta = e / jnp.sum(e)
        beta_ref[0] = beta.reshape(1, S)
        z = jnp.dot(beta.reshape(1, S), x, preferred_element_type=jnp.float32)
        z = jnp.where(z >= 0, z, 0.01 * z)
        z_ref[0] = jnp.tanh(z)

    z, beta = pl.pallas_call(
        body,
        grid=(E,),
        in_specs=[
            pl.BlockSpec((1, S, D), lambda e: (e, 0, 0)),
            pl.BlockSpec((1, D, H), lambda e: (e, 0, 0)),
            pl.BlockSpec((1, 1, H), lambda e: (e, 0, 0)),
            pl.BlockSpec((1, 1, H), lambda e: (e, 0, 0)),
        ],
        out_specs=[
            pl.BlockSpec((1, 1, D), lambda e: (e, 0, 0)),
            pl.BlockSpec((1, 1, S), lambda e: (e, 0, 0)),
        ],
        out_shape=[
            jax.ShapeDtypeStruct((E, 1, D), jnp.float32),
            jax.ShapeDtypeStruct((E, 1, S), jnp.float32),
        ],
    )(xg, W1, b1, W2v)
    return z.reshape(E, D), beta.reshape(E, S)


def kernel(X, hyperedge_index, W1, b1, W2, b2):
    E, S = hyperedge_index.shape
    D = X.shape[1]
    idx_flat = hyperedge_index.reshape(-1).astype(jnp.int32)
    xg = _sc_gather(X, idx_flat).reshape(E, S, D)
    z, beta = _tc_attention(xg, W1, b1, W2[..., 0])
    return z, beta[..., None]
